# baseline (speedup 1.0000x reference)
import functools

import jax
import jax.numpy as jnp
from jax.experimental import pallas as pl
from jax.experimental.pallas import tpu as pltpu

_LANE = 128


def _dice_kernel(o_ref, l_ref, out_ref, acc_ref, *, nb, c, steps):
    j = pl.program_id(0)

    @pl.when(j == 0)
    def _init():
        acc_ref[...] = jnp.zeros_like(acc_ref)

    acc = jnp.float32(0.0)
    for ni in range(nb):
        for ci in range(c):
            o = o_ref[ni, ci].astype(jnp.float32)
            l = l_ref[ni, ci].astype(jnp.float32)
            num = jnp.sum(o * l)
            den = jnp.sum(o * o + l)
            acc += (num + num) / den
    acc_ref[...] += acc

    @pl.when(j == steps - 1)
    def _finalize():
        out_ref[...] = 1.0 - 0.5 * acc_ref[...] / (nb * c * steps)


def kernel(outputs, labels):
    n, c, h, w = outputs.shape
    nb = 1
    steps = n // nb

    body = functools.partial(_dice_kernel, nb=nb, c=c, steps=steps)

    out = pl.pallas_call(
        body,
        out_shape=jax.ShapeDtypeStruct((1, 1), jnp.float32),
        grid_spec=pltpu.PrefetchScalarGridSpec(
            num_scalar_prefetch=0,
            grid=(steps,),
            in_specs=[
                pl.BlockSpec((nb, c, h, w), lambda j: (j, 0, 0, 0)),
                pl.BlockSpec((nb, c, h, w), lambda j: (j, 0, 0, 0)),
            ],
            out_specs=pl.BlockSpec((1, 1), lambda j: (0, 0)),
            scratch_shapes=[pltpu.VMEM((1, 1), jnp.float32)],
        ),
        compiler_params=pltpu.CompilerParams(
            dimension_semantics=("arbitrary",),
            vmem_limit_bytes=48 * 1024 * 1024,
        ),
    )(outputs, labels)

    return out[0, 0]


# speedup vs baseline: 1.0924x; 1.0924x over previous
import functools

import jax
import jax.numpy as jnp
from jax.experimental import pallas as pl
from jax.experimental.pallas import tpu as pltpu

_LANE = 128


def _dice_kernel(o_ref, l_ref, out_ref, acc_ref, *, nb, c, steps):
    j = pl.program_id(0)

    @pl.when(j == 0)
    def _init():
        acc_ref[...] = jnp.zeros_like(acc_ref)

    acc = jnp.float32(0.0)
    for ni in range(nb):
        for ci in range(c):
            o = o_ref[ni, ci].astype(jnp.float32)
            l = l_ref[ni, ci].astype(jnp.float32)
            num = jnp.sum(o * l)
            den = jnp.sum(o * o + l)
            acc += (num + num) / den
    acc_ref[...] += acc

    @pl.when(j == steps - 1)
    def _finalize():
        out_ref[...] = 1.0 - 0.5 * acc_ref[...] / (nb * c * steps)


def kernel(outputs, labels):
    n, c, h, w = outputs.shape
    nb = 4 if n % 4 == 0 else (2 if n % 2 == 0 else 1)
    steps = n // nb

    body = functools.partial(_dice_kernel, nb=nb, c=c, steps=steps)

    out = pl.pallas_call(
        body,
        out_shape=jax.ShapeDtypeStruct((1, 1), jnp.float32),
        grid_spec=pltpu.PrefetchScalarGridSpec(
            num_scalar_prefetch=0,
            grid=(steps,),
            in_specs=[
                pl.BlockSpec((nb, c, h, w), lambda j: (j, 0, 0, 0)),
                pl.BlockSpec((nb, c, h, w), lambda j: (j, 0, 0, 0)),
            ],
            out_specs=pl.BlockSpec((1, 1), lambda j: (0, 0)),
            scratch_shapes=[pltpu.VMEM((1, 1), jnp.float32)],
        ),
        compiler_params=pltpu.CompilerParams(
            dimension_semantics=("arbitrary",),
            vmem_limit_bytes=48 * 1024 * 1024,
        ),
    )(outputs, labels)

    return out[0, 0]


# speedup vs baseline: 1.1510x; 1.0536x over previous
import functools

import jax
import jax.numpy as jnp
from jax.experimental import pallas as pl
from jax.experimental.pallas import tpu as pltpu

_LANE = 128


def _dice_kernel(o_ref, l_ref, out_ref, acc_ref, *, nb, c, steps):
    j = pl.program_id(0)

    @pl.when(j == 0)
    def _init():
        acc_ref[...] = jnp.zeros_like(acc_ref)

    acc = jnp.float32(0.0)
    for ni in range(nb):
        for ci in range(c):
            o = o_ref[ni, ci].astype(jnp.float32)
            l = l_ref[ni, ci].astype(jnp.float32)
            num = jnp.sum(o * l)
            den = jnp.sum(o * o + l)
            acc += (num + num) / den
    acc_ref[...] += acc

    @pl.when(j == steps - 1)
    def _finalize():
        out_ref[...] = 1.0 - 0.5 * acc_ref[...] / (nb * c * steps)


def kernel(outputs, labels):
    n, c, h, w = outputs.shape
    nb = 2 if n % 2 == 0 else 1
    steps = n // nb

    body = functools.partial(_dice_kernel, nb=nb, c=c, steps=steps)

    out = pl.pallas_call(
        body,
        out_shape=jax.ShapeDtypeStruct((1, 1), jnp.float32),
        grid_spec=pltpu.PrefetchScalarGridSpec(
            num_scalar_prefetch=0,
            grid=(steps,),
            in_specs=[
                pl.BlockSpec((nb, c, h, w), lambda j: (j, 0, 0, 0)),
                pl.BlockSpec((nb, c, h, w), lambda j: (j, 0, 0, 0)),
            ],
            out_specs=pl.BlockSpec((1, 1), lambda j: (0, 0)),
            scratch_shapes=[pltpu.VMEM((1, 1), jnp.float32)],
        ),
        compiler_params=pltpu.CompilerParams(
            dimension_semantics=("arbitrary",),
            vmem_limit_bytes=48 * 1024 * 1024,
        ),
    )(outputs, labels)

    return out[0, 0]
